# baseline (speedup 1.0000x reference)
import functools

import jax
import jax.numpy as jnp
from jax import lax
from jax.experimental import pallas as pl
from jax.experimental.pallas import tpu as pltpu
from jax.experimental.pallas import tpu_sc as plsc

NUM_OBS = 1000
NUM_ACT = 100
VOCAB = NUM_OBS * NUM_ACT
D = 32
B = 4096

NC = 2
NS = 16
L = 16
NW = NC * NS

TILES_PER_W = 26
CPW = TILES_PER_W * 128
HALF = CPW // 2
LAST_TILE = (VOCAB + 127) // 128 - TILES_PER_W
MAX_HITS = 128
N_GROUPS = MAX_HITS // L


_sc_mesh = plsc.VectorSubcoreMesh(
    core_axis_name="c", subcore_axis_name="s", num_cores=NC, num_subcores=NS
)


@functools.partial(
    pl.kernel,
    out_type=(
        jax.ShapeDtypeStruct((B, D), jnp.float32),
        jax.ShapeDtypeStruct((B, D), jnp.float32),
    ),
    mesh=_sc_mesh,
    compiler_params=pltpu.CompilerParams(needs_layout_passes=False),
    scratch_types=[
        pltpu.VMEM((B,), jnp.int32),
        pltpu.VMEM((B,), jnp.int32),
        pltpu.VMEM((B,), jnp.int32),
        pltpu.VMEM((D, HALF), jnp.float32),
        pltpu.VMEM((MAX_HITS + L,), jnp.int32),
        pltpu.VMEM((MAX_HITS + L,), jnp.int32),
        pltpu.VMEM((MAX_HITS, D), jnp.float32),
        pltpu.VMEM((1, D), jnp.float32),
        pltpu.SemaphoreType.DMA,
    ],
)
def _sc_gather(obs_hbm, act_hbm, fobs_hbm, fact_hbm, wft_hbm, wbt_hbm,
               fwd_hbm, bwd_hbm,
               idxf_v, idxb_v, tmp_v, stripe, ccol_v, cpos_v, stage,
               scrap, sem):
    wid = lax.axis_index("s") * NC + lax.axis_index("c")
    tile0 = jnp.minimum(wid * TILES_PER_W, LAST_TILE)
    lo = pl.multiple_of(tile0 * 128, 128)

    pltpu.sync_copy(obs_hbm, idxf_v)
    pltpu.sync_copy(act_hbm, tmp_v)
    for i in range(B // L):
        v = pl.ds(i * L, L)
        idxf_v[v] = (jnp.clip(idxf_v[v], 0, NUM_OBS - 1) * NUM_ACT
                     + jnp.clip(tmp_v[v], 0, NUM_ACT - 1))
    pltpu.sync_copy(fobs_hbm, idxb_v)
    pltpu.sync_copy(fact_hbm, tmp_v)
    for i in range(B // L):
        v = pl.ds(i * L, L)
        idxb_v[v] = (jnp.clip(idxb_v[v], 0, NUM_OBS - 1) * NUM_ACT
                     + jnp.clip(tmp_v[v], 0, NUM_ACT - 1))

    lane = jax.lax.iota(jnp.int32, L)
    for i in range((MAX_HITS + L) // L):
        ccol_v[pl.ds(i * L, L)] = jnp.zeros((L,), jnp.int32)
        cpos_v[pl.ds(i * L, L)] = jnp.zeros((L,), jnp.int32)

    def one_pass(wt_hbm, out_hbm, idx_v, plo):
        pltpu.sync_copy(wt_hbm.at[:, pl.ds(plo, HALF)], stripe)
        phi = plo + HALF

        def scan(i, cnt):
            ch = idx_v[pl.ds(i * L, L)]
            m = jnp.logical_and(ch >= plo, ch < phi)
            plsc.store_compressed(ccol_v.at[pl.ds(cnt, L)], ch - plo, mask=m)
            plsc.store_compressed(cpos_v.at[pl.ds(cnt, L)], i * L + lane, mask=m)
            npop = plsc.all_reduce_population_count(m)
            return cnt + lax.reduce_max(npop, (0,))

        nh = lax.fori_loop(0, B // L, scan, jnp.int32(0))

        def group(g, carry):
            gbase = g * L
            cvec = ccol_v[pl.ds(gbase, L)]
            pvec = cpos_v[pl.ds(gbase, L)]
            lvec = gbase + lane
            for d in range(D):
                dvec = jnp.full((L,), d, jnp.int32)
                vals = plsc.load_gather(stripe, [dvec, cvec])
                plsc.store_scatter(stage, [lvec, dvec], vals)
            for l in range(L):
                pos = pl.multiple_of(
                    jnp.sum(jnp.where(lane == l, pvec, 0)), 1)
                live = gbase + l < nh

                @pl.when(live)
                def _():
                    pltpu.async_copy(stage.at[pl.ds(gbase + l, 1)],
                                     out_hbm.at[pl.ds(pos, 1)], sem)

                @pl.when(jnp.logical_not(live))
                def _():
                    pltpu.async_copy(out_hbm.at[pl.ds(0, 1)], scrap, sem)
            return carry

        lax.fori_loop(0, N_GROUPS, group, 0)
        pltpu.make_async_copy(
            out_hbm.at[pl.ds(0, MAX_HITS)], stage, sem).wait()

    for wt, out, idx in ((wft_hbm, fwd_hbm, idxf_v), (wbt_hbm, bwd_hbm, idxb_v)):
        for h in range(2):
            one_pass(wt, out, idx, pl.multiple_of(lo + h * HALF, 128))


def _mm_body(a_ref, b_ref, o_ref):
    o_ref[...] = lax.dot_general(
        a_ref[...], b_ref[...],
        (((1,), (1,)), ((), ())),
        preferred_element_type=jnp.float32,
    )


_ROWS_PER_STEP = 512


def _matmul(fwd, bwd):
    return pl.pallas_call(
        _mm_body,
        grid=(B // _ROWS_PER_STEP,),
        in_specs=[
            pl.BlockSpec((_ROWS_PER_STEP, D), lambda i: (i, 0)),
            pl.BlockSpec((B, D), lambda i: (0, 0)),
        ],
        out_specs=pl.BlockSpec((_ROWS_PER_STEP, B), lambda i: (i, 0)),
        out_shape=jax.ShapeDtypeStruct((B, B), jnp.float32),
    )(fwd, bwd)


def kernel(observations, actions, future_observations, future_actions,
           W_forward, W_backward):
    obs = observations.astype(jnp.int32)
    act = actions.astype(jnp.int32)
    fobs = future_observations.astype(jnp.int32)
    fact = future_actions.astype(jnp.int32)
    fwd, bwd = _sc_gather(obs, act, fobs, fact, W_forward.T, W_backward.T)
    return _matmul(fwd, bwd)


# speedup vs baseline: 3.2612x; 3.2612x over previous
import functools

import jax
import jax.numpy as jnp
from jax import lax
from jax.experimental import pallas as pl
from jax.experimental.pallas import tpu as pltpu
from jax.experimental.pallas import tpu_sc as plsc

NUM_OBS = 1000
NUM_ACT = 100
VOCAB = NUM_OBS * NUM_ACT
D = 32
B = 4096

NC = 2
NS = 16
L = 16
NW = NC * NS
B_PER_W = B // NW


_sc_mesh = plsc.VectorSubcoreMesh(
    core_axis_name="c", subcore_axis_name="s", num_cores=NC, num_subcores=NS
)


@functools.partial(
    pl.kernel,
    out_type=(
        jax.ShapeDtypeStruct((B, D), jnp.float32),
        jax.ShapeDtypeStruct((B, D), jnp.float32),
    ),
    mesh=_sc_mesh,
    compiler_params=pltpu.CompilerParams(needs_layout_passes=False),
    scratch_types=[
        pltpu.VMEM((B_PER_W,), jnp.int32),
        pltpu.VMEM((B_PER_W,), jnp.int32),
        pltpu.VMEM((B_PER_W,), jnp.int32),
        pltpu.VMEM((B_PER_W,), jnp.int32),
        pltpu.VMEM((B_PER_W, D), jnp.float32),
        pltpu.VMEM((B_PER_W, D), jnp.float32),
        pltpu.SemaphoreType.DMA,
        pltpu.SemaphoreType.DMA,
    ],
)
def _sc_gather(obs_hbm, act_hbm, fobs_hbm, fact_hbm, wf_hbm, wb_hbm,
               fwd_hbm, bwd_hbm,
               idxf_v, idxb_v, actf_v, actb_v, rows_f, rows_b, semf, semb):
    wid = lax.axis_index("s") * NC + lax.axis_index("c")
    base = wid * B_PER_W
    sl = pl.ds(base, B_PER_W)
    pltpu.sync_copy(obs_hbm.at[sl], idxf_v)
    pltpu.sync_copy(act_hbm.at[sl], actf_v)
    pltpu.sync_copy(fobs_hbm.at[sl], idxb_v)
    pltpu.sync_copy(fact_hbm.at[sl], actb_v)
    for i in range(B_PER_W // L):
        v = pl.ds(i * L, L)
        idxf_v[v] = (jnp.clip(idxf_v[v], 0, NUM_OBS - 1) * NUM_ACT
                     + jnp.clip(actf_v[v], 0, NUM_ACT - 1))
        idxb_v[v] = (jnp.clip(idxb_v[v], 0, NUM_OBS - 1) * NUM_ACT
                     + jnp.clip(actb_v[v], 0, NUM_ACT - 1))
    lane = jax.lax.iota(jnp.int32, L)

    def fire(j, carry):
        c = (j // L) * L
        k = j - c
        chf = idxf_v[pl.ds(c, L)]
        chb = idxb_v[pl.ds(c, L)]
        rf = jnp.sum(jnp.where(lane == k, chf, 0))
        rb = jnp.sum(jnp.where(lane == k, chb, 0))
        pltpu.async_copy(wf_hbm.at[pl.ds(rf, 1)], rows_f.at[pl.ds(j, 1)], semf)
        pltpu.async_copy(wb_hbm.at[pl.ds(rb, 1)], rows_b.at[pl.ds(j, 1)], semb)
        return carry

    lax.fori_loop(0, B_PER_W, fire, 0)
    pltpu.make_async_copy(wf_hbm.at[pl.ds(0, B_PER_W)], rows_f, semf).wait()
    pltpu.make_async_copy(wb_hbm.at[pl.ds(0, B_PER_W)], rows_b, semb).wait()
    pltpu.sync_copy(rows_f, fwd_hbm.at[sl])
    pltpu.sync_copy(rows_b, bwd_hbm.at[sl])


_TP_COLS = 2048
_TP_GRID = (VOCAB + _TP_COLS - 1) // _TP_COLS


def _tp_body(af_ref, ab_ref, of_ref, ob_ref):
    of_ref[...] = af_ref[...].T
    ob_ref[...] = ab_ref[...].T


def _transpose_tables(wft, wbt):
    return pl.pallas_call(
        _tp_body,
        grid=(_TP_GRID,),
        in_specs=[
            pl.BlockSpec((D, _TP_COLS), lambda i: (0, i)),
            pl.BlockSpec((D, _TP_COLS), lambda i: (0, i)),
        ],
        out_specs=[
            pl.BlockSpec((_TP_COLS, D), lambda i: (i, 0)),
            pl.BlockSpec((_TP_COLS, D), lambda i: (i, 0)),
        ],
        out_shape=(
            jax.ShapeDtypeStruct((VOCAB, D), jnp.float32),
            jax.ShapeDtypeStruct((VOCAB, D), jnp.float32),
        ),
    )(wft, wbt)


def _mm_body(a_ref, b_ref, o_ref):
    o_ref[...] = lax.dot_general(
        a_ref[...], b_ref[...],
        (((1,), (1,)), ((), ())),
        preferred_element_type=jnp.float32,
    )


_ROWS_PER_STEP = 512


def _matmul(fwd, bwd):
    return pl.pallas_call(
        _mm_body,
        grid=(B // _ROWS_PER_STEP,),
        in_specs=[
            pl.BlockSpec((_ROWS_PER_STEP, D), lambda i: (i, 0)),
            pl.BlockSpec((B, D), lambda i: (0, 0)),
        ],
        out_specs=pl.BlockSpec((_ROWS_PER_STEP, B), lambda i: (i, 0)),
        out_shape=jax.ShapeDtypeStruct((B, B), jnp.float32),
    )(fwd, bwd)


def kernel(observations, actions, future_observations, future_actions,
           W_forward, W_backward):
    obs = observations.astype(jnp.int32)
    act = actions.astype(jnp.int32)
    fobs = future_observations.astype(jnp.int32)
    fact = future_actions.astype(jnp.int32)
    wf_rm, wb_rm = _transpose_tables(W_forward.T, W_backward.T)
    fwd, bwd = _sc_gather(obs, act, fobs, fact, wf_rm, wb_rm)
    return _matmul(fwd, bwd)
